# baseline (speedup 1.0000x reference)
import functools

import jax
import jax.numpy as jnp
from jax import lax
from jax.experimental import pallas as pl
from jax.experimental.pallas import tpu as pltpu
from jax.experimental.pallas import tpu_sc as plsc

N = 10000
E = 320000
ET = E + N
D_IN = 128
HID = 64
HEADS = 8
HD = HEADS * HID
NHP = 4
NP = 10240
EP = 331776
NC = 2
NS = 16
KB = 128
RPT = NP // NS

BN = 256
NBLK = NP // BN

def _mesh():
    return plsc.VectorSubcoreMesh(
        core_axis_name="c", subcore_axis_name="s",
        num_cores=NC, num_subcores=NS)


_SC_PARAMS = pltpu.CompilerParams(use_tc_tiling_on_sc=False,
                                  needs_layout_passes=False)


_f32 = jnp.float32


def _zero_vmem(ref, nrow, ncol):
    zv = jnp.zeros((16,), _f32)

    def body(r, _):
        for j in range(ncol // 16):
            ref[r, pl.ds(j * 16, 16)] = zv
        return 0

    lax.fori_loop(0, nrow, body, 0)


_KW_SB = 3
_KW_NSB = EP // (NC * NS) // (_KW_SB * KB)
_KW_NPAIR = _KW_NSB // 2


def _kw_core(want_deg, ts_hbm, td_hbm, src2_hbm, dst2_hbm, w_hbm, den_hbm,
             deg_hbm, is_a, id_a, is_b, id_b, rs_a, rd_a, rs_b, rd_b,
             wv_a, wv_b, ones, den_sh, deg_sh, sem_a, sem_b):
    cid = lax.axis_index("c")
    sid = lax.axis_index("s")
    row0 = sid * RPT
    SB = _KW_SB
    tile_blk0 = (cid * NS + sid) * _KW_NSB * SB

    zb = wv_a.at[0]
    _zero_vmem(zb, KB, 16)
    for i in range(RPT // KB):
        pltpu.sync_copy(zb, den_sh.at[pl.ds(row0 + i * KB, KB)])
        if want_deg:
            pltpu.sync_copy(zb, deg_sh.at[pl.ds(row0 + i * KB, KB)])
    if want_deg:
        ov = jnp.full((16,), 1.0, _f32)

        def fill1(r, _):
            ones[r, pl.ds(0, 16)] = ov
            return 0

        lax.fori_loop(0, KB, fill1, 0)
    plsc.subcore_barrier()

    def fire(s, is2, id2, rs3, rd3, sem):
        brow = tile_blk0 + s * SB
        pltpu.sync_copy(src2_hbm.at[pl.ds(brow, SB)], is2)
        pltpu.sync_copy(dst2_hbm.at[pl.ds(brow, SB)], id2)
        for j in range(SB):
            pltpu.async_copy(ts_hbm.at[is2.at[j]], rs3.at[j], sem)
            pltpu.async_copy(td_hbm.at[id2.at[j]], rd3.at[j], sem)

    def drain(is2, id2, rs3, rd3, sem):
        for j in range(SB):
            pltpu.make_async_copy(ts_hbm.at[is2.at[j]], rs3.at[j], sem).wait()
            pltpu.make_async_copy(td_hbm.at[id2.at[j]], rd3.at[j], sem).wait()

    def proc(s, id2, rs3, rd3, wv3):
        brow = tile_blk0 + s * SB
        for j in range(SB):
            rsj = rs3.at[j]
            rdj = rd3.at[j]
            wvj = wv3.at[j]

            @plsc.parallel_loop(0, KB, unroll=8)
            def _(k):
                e = rsj[k, pl.ds(0, 16)] + rdj[k, pl.ds(0, 16)]
                e = jnp.where(e > 0, e, 0.2 * e)
                wvj[k, pl.ds(0, 16)] = jnp.exp(e)

            pltpu.sync_copy(wvj, w_hbm.at[pl.ds((brow + j) * KB, KB)])
            pltpu.sync_copy(wvj, den_sh.at[id2.at[j]], add=True)
            if want_deg:
                pltpu.sync_copy(ones, deg_sh.at[id2.at[j]], add=True)

    fire(0, is_a, id_a, rs_a, rd_a, sem_a)

    def pair(p, _):
        s0 = 2 * p
        fire(s0 + 1, is_b, id_b, rs_b, rd_b, sem_b)
        drain(is_a, id_a, rs_a, rd_a, sem_a)
        proc(s0, id_a, rs_a, rd_a, wv_a)
        fire(s0 + 2, is_a, id_a, rs_a, rd_a, sem_a)
        drain(is_b, id_b, rs_b, rd_b, sem_b)
        proc(s0 + 1, id_b, rs_b, rd_b, wv_b)
        return 0

    lax.fori_loop(0, _KW_NPAIR, pair, 0)
    drain(is_a, id_a, rs_a, rd_a, sem_a)
    proc(_KW_NSB - 1, id_a, rs_a, rd_a, wv_a)

    plsc.subcore_barrier()
    pltpu.sync_copy(den_sh.at[pl.ds(row0, RPT)],
                    den_hbm.at[cid].at[pl.ds(row0, RPT)])
    if want_deg:
        pltpu.sync_copy(deg_sh.at[pl.ds(row0, RPT)],
                        deg_hbm.at[cid].at[pl.ds(row0, RPT)])


def _kw_body_deg(*args):
    _kw_core(True, *args)


def _kw_body_nodeg(ts_hbm, td_hbm, src2_hbm, dst2_hbm, w_hbm, den_hbm,
                   *rest):
    _kw_core(False, ts_hbm, td_hbm, src2_hbm, dst2_hbm, w_hbm, den_hbm,
             None, *rest)


@functools.lru_cache(maxsize=None)
def _build_kw(want_deg):
    out_type = [
        jax.ShapeDtypeStruct((EP, 16), _f32),
        jax.ShapeDtypeStruct((NC, NP, 16), _f32),
    ]
    if want_deg:
        out_type.append(jax.ShapeDtypeStruct((NC, NP, 16), _f32))
    return pl.kernel(
        _kw_body_deg if want_deg else _kw_body_nodeg,
        out_type=out_type,
        mesh=_mesh(),
        compiler_params=_SC_PARAMS,
        scratch_types=[
            pltpu.VMEM((_KW_SB, KB), jnp.int32),
            pltpu.VMEM((_KW_SB, KB), jnp.int32),
            pltpu.VMEM((_KW_SB, KB), jnp.int32),
            pltpu.VMEM((_KW_SB, KB), jnp.int32),
            pltpu.VMEM((_KW_SB, KB, 16), _f32),
            pltpu.VMEM((_KW_SB, KB, 16), _f32),
            pltpu.VMEM((_KW_SB, KB, 16), _f32),
            pltpu.VMEM((_KW_SB, KB, 16), _f32),
            pltpu.VMEM((_KW_SB, KB, 16), _f32),
            pltpu.VMEM((_KW_SB, KB, 16), _f32),
            pltpu.VMEM((KB, 16), _f32),
            pltpu.VMEM_SHARED((NP, 16), _f32),
            pltpu.VMEM_SHARED((NP, 16), _f32),
            pltpu.SemaphoreType.DMA,
            pltpu.SemaphoreType.DMA,
        ],
    )


def _kw(ts, td, src2, dst2):
    return _build_kw(True)(ts, td, src2, dst2)


def _kw_nodeg(ts, td, src2, dst2):
    return _build_kw(False)(ts, td, src2, dst2)


_AGG_NBLK = EP // NS // KB
_AGG_NPAIR = _AGG_NBLK // 2


def _kagg_body(h_hbm, w_hbm, src_hbm, dst_hbm, num_hbm,
               is_a, id_a, is_b, id_b, rows_a, rows_b, wv_a, wv_b, acc,
               sem_a, sem_b, sem_sa, sem_sb):
    cid = lax.axis_index("c")
    sid = lax.axis_index("s")
    nblk = _AGG_NBLK
    row0 = sid * RPT
    blk0 = sid * nblk

    for hpi in range(2):
        _zero_vmem(rows_a, KB, 128)
        for i in range(RPT // KB):
            pltpu.sync_copy(rows_a, acc.at[pl.ds(row0 + i * KB, KB)])
        plsc.subcore_barrier()

        h_hp = h_hbm.at[2 * cid + hpi]
        num_hp = num_hbm.at[2 * cid + hpi]
        col0 = jnp.full((16,), 4 * cid + 2 * hpi, jnp.int32)
        col1 = col0 + 1

        def fire(b, is_r, id_r, rows_r, wv_r, sem):
            base = (blk0 + b) * KB
            pltpu.sync_copy(src_hbm.at[pl.ds(base, KB)], is_r)
            pltpu.sync_copy(dst_hbm.at[pl.ds(base, KB)], id_r)
            pltpu.async_copy(h_hp.at[is_r], rows_r, sem)
            pltpu.async_copy(w_hbm.at[pl.ds(base, KB)], wv_r, sem)

        def drain(is_r, rows_r, wv_r, sem):
            pltpu.make_async_copy(h_hp.at[is_r], rows_r, sem).wait()
            pltpu.make_async_copy(w_hbm.at[pl.ds(0, KB)], wv_r, sem).wait()

        def scale(rows_r, wv_r):
            @plsc.parallel_loop(0, KB, unroll=8)
            def _(k):
                kvec = jnp.full((16,), k, jnp.int32)
                w0 = plsc.load_gather(wv_r, [kvec, col0])
                w1 = plsc.load_gather(wv_r, [kvec, col1])
                for j in range(4):
                    rows_r[k, pl.ds(j * 16, 16)] = (
                        rows_r[k, pl.ds(j * 16, 16)] * w0)
                for j in range(4, 8):
                    rows_r[k, pl.ds(j * 16, 16)] = (
                        rows_r[k, pl.ds(j * 16, 16)] * w1)

        def fire_scat(id_r, rows_r, sem):
            pltpu.async_copy(rows_r, acc.at[id_r], add=True, sem=sem)

        def wait_scat(id_r, rows_r, sem):
            pltpu.make_async_copy(rows_r, acc.at[id_r], sem).wait()

        fire(0, is_a, id_a, rows_a, wv_a, sem_a)

        def pair(p, _):
            b0 = 2 * p

            @pl.when(p > 0)
            def _():
                wait_scat(id_b, rows_b, sem_sb)

            fire(b0 + 1, is_b, id_b, rows_b, wv_b, sem_b)
            drain(is_a, rows_a, wv_a, sem_a)
            scale(rows_a, wv_a)
            fire_scat(id_a, rows_a, sem_sa)

            drain(is_b, rows_b, wv_b, sem_b)
            scale(rows_b, wv_b)

            wait_scat(id_a, rows_a, sem_sa)

            @pl.when(p < _AGG_NPAIR - 1)
            def _():
                fire(b0 + 2, is_a, id_a, rows_a, wv_a, sem_a)

            fire_scat(id_b, rows_b, sem_sb)
            return 0

        lax.fori_loop(0, _AGG_NPAIR, pair, 0)
        wait_scat(id_b, rows_b, sem_sb)
        plsc.subcore_barrier()
        pltpu.sync_copy(acc.at[pl.ds(row0, RPT)], num_hp.at[pl.ds(row0, RPT)])
        plsc.subcore_barrier()


@functools.lru_cache(maxsize=None)
def _build_kagg():
    return pl.kernel(
        _kagg_body,
        out_type=jax.ShapeDtypeStruct((NHP, NP, 128), _f32),
        mesh=_mesh(),
        compiler_params=_SC_PARAMS,
        scratch_types=[
            pltpu.VMEM((KB,), jnp.int32),
            pltpu.VMEM((KB,), jnp.int32),
            pltpu.VMEM((KB,), jnp.int32),
            pltpu.VMEM((KB,), jnp.int32),
            pltpu.VMEM((KB, 128), _f32),
            pltpu.VMEM((KB, 128), _f32),
            pltpu.VMEM((KB, 16), _f32),
            pltpu.VMEM((KB, 16), _f32),
            pltpu.VMEM_SHARED((NP, 128), _f32),
            pltpu.SemaphoreType.DMA,
            pltpu.SemaphoreType.DMA,
            pltpu.SemaphoreType.DMA,
            pltpu.SemaphoreType.DMA,
        ],
    )


def _kagg(*args):
    return _build_kagg()(*args)


def _kgcn_body(h_hbm, src2_hbm, dst2_hbm, out_hbm,
               is_a, id_a, is_b, id_b, rows_a, rows_b, acc, sem_a, sem_b):
    cid = lax.axis_index("c")
    sid = lax.axis_index("s")
    row0 = sid * RPT
    SB = _KW_SB
    tile_blk0 = (cid * NS + sid) * _KW_NSB * SB

    zb = rows_a.at[0]
    _zero_vmem(zb, KB, 64)
    for i in range(RPT // KB):
        pltpu.sync_copy(zb, acc.at[pl.ds(row0 + i * KB, KB)])
    plsc.subcore_barrier()

    def fire(s, is2, id2, rows3, sem):
        brow = tile_blk0 + s * SB
        pltpu.sync_copy(src2_hbm.at[pl.ds(brow, SB)], is2)
        pltpu.sync_copy(dst2_hbm.at[pl.ds(brow, SB)], id2)
        for j in range(SB):
            pltpu.async_copy(h_hbm.at[is2.at[j]], rows3.at[j], sem)

    def drain(is2, rows3, sem):
        for j in range(SB):
            pltpu.make_async_copy(h_hbm.at[is2.at[j]], rows3.at[j],
                                  sem).wait()

    def proc(id2, rows3):
        for j in range(SB):
            pltpu.sync_copy(rows3.at[j], acc.at[id2.at[j]], add=True)

    fire(0, is_a, id_a, rows_a, sem_a)

    def pair(p, _):
        fire(2 * p + 1, is_b, id_b, rows_b, sem_b)
        drain(is_a, rows_a, sem_a)
        proc(id_a, rows_a)
        fire(2 * p + 2, is_a, id_a, rows_a, sem_a)
        drain(is_b, rows_b, sem_b)
        proc(id_b, rows_b)
        return 0

    lax.fori_loop(0, _KW_NPAIR, pair, 0)
    drain(is_a, rows_a, sem_a)
    proc(id_a, rows_a)

    plsc.subcore_barrier()
    pltpu.sync_copy(acc.at[pl.ds(row0, RPT)],
                    out_hbm.at[cid].at[pl.ds(row0, RPT)])


@functools.lru_cache(maxsize=None)
def _build_kgcn():
    return pl.kernel(
        _kgcn_body,
        out_type=jax.ShapeDtypeStruct((NC, NP, 64), _f32),
        mesh=_mesh(),
        compiler_params=_SC_PARAMS,
        scratch_types=[
            pltpu.VMEM((_KW_SB, KB), jnp.int32),
            pltpu.VMEM((_KW_SB, KB), jnp.int32),
            pltpu.VMEM((_KW_SB, KB), jnp.int32),
            pltpu.VMEM((_KW_SB, KB), jnp.int32),
            pltpu.VMEM((_KW_SB, KB, 64), _f32),
            pltpu.VMEM((_KW_SB, KB, 64), _f32),
            pltpu.VMEM_SHARED((NP, 64), _f32),
            pltpu.SemaphoreType.DMA,
            pltpu.SemaphoreType.DMA,
        ],
    )


def _kgcn(*args):
    return _build_kgcn()(*args)


_DOT = functools.partial(jnp.dot, preferred_element_type=_f32,
                         precision=jax.lax.Precision.HIGHEST)


def _p0_body(x_ref, w1_ref, as_ref, ad_ref, h_ref, ts_ref, td_ref):
    xb = x_ref[...]
    hs = []
    for hp in range(NHP):
        h = _DOT(xb, w1_ref[hp])
        h_ref[hp] = h
        hs.append(h)
    ts_ref[...] = sum(_DOT(hs[hp], as_ref[hp]) for hp in range(NHP))
    td_ref[...] = sum(_DOT(hs[hp], ad_ref[hp]) for hp in range(NHP))


def _norm_elu(num_ref, den_ref, b_ref, g_ref, be_ref, hp):
    den = den_ref[0] + den_ref[1]
    r0 = 1.0 / (den[:, 2 * hp:2 * hp + 1] + 1e-16)
    r1 = 1.0 / (den[:, 2 * hp + 1:2 * hp + 2] + 1e-16)
    numb = num_ref[hp]
    y = jnp.concatenate([numb[:, 0:64] * r0, numb[:, 64:128] * r1], axis=1)
    y = y + b_ref[hp][None, :]
    y = y * (g_ref[hp][None, :] / jnp.sqrt(1.0 + 1e-5)) + be_ref[hp][None, :]
    return jnp.where(y > 0, y, jnp.exp(y) - 1.0)


def _p3_body(num_ref, den_ref, b_ref, g_ref, be_ref, w2_ref, as_ref, ad_ref,
             h2_ref, ts_ref, td_ref):
    ys = [_norm_elu(num_ref, den_ref, b_ref, g_ref, be_ref, hp)
          for hp in range(NHP)]
    for ohp in range(NHP):
        h2_ref[ohp] = sum(_DOT(ys[hp], w2_ref[hp, ohp]) for hp in range(NHP))
    ts_ref[...] = sum(_DOT(ys[hp], as_ref[hp]) for hp in range(NHP))
    td_ref[...] = sum(_DOT(ys[hp], ad_ref[hp]) for hp in range(NHP))


def _p6_body(num_ref, den_ref, b_ref, g_ref, be_ref, w3_ref, deg_ref, h3_ref):
    ys = [_norm_elu(num_ref, den_ref, b_ref, g_ref, be_ref, hp)
          for hp in range(NHP)]
    h3 = sum(_DOT(ys[hp], w3_ref[hp]) for hp in range(NHP))
    deg = (deg_ref[0] + deg_ref[1])[:, 0:1]
    dinv = jax.lax.rsqrt(jnp.maximum(deg, 1.0))
    h3_ref[...] = h3 * dinv


def _p8_body(gcn_ref, deg_ref, b3_ref, g3_ref, be3_ref, wl_ref, bl_ref,
             out_ref):
    deg = (deg_ref[0] + deg_ref[1])[:, 0:1]
    dinv = jax.lax.rsqrt(jnp.maximum(deg, 1.0))
    y = (gcn_ref[0] + gcn_ref[1]) * dinv + b3_ref[...][None, :]
    y = y * (g3_ref[...][None, :] / jnp.sqrt(1.0 + 1e-5)) + be3_ref[...][None, :]
    y = jnp.where(y > 0, y, jnp.exp(y) - 1.0)
    logits = _DOT(y, wl_ref[...]) + bl_ref[...][None, :]
    l0 = logits[:, 0:1]
    l1 = logits[:, 1:2]
    m = jnp.maximum(l0, l1)
    lse = m + jnp.log(jnp.exp(l0 - m) + jnp.exp(l1 - m))
    out_ref[...] = jnp.concatenate([l0 - lse, l1 - lse], axis=1)


def _row_spec(shape):
    nd = len(shape)
    return pl.BlockSpec((BN,) + shape[1:], lambda i: (i,) + (0,) * (nd - 1))


def _full_spec(shape):
    nd = len(shape)
    return pl.BlockSpec(shape, lambda i: (0,) * nd)


def _lead_row_spec(shape):
    nd = len(shape)
    return pl.BlockSpec((shape[0], BN) + shape[2:],
                        lambda i: (0, i) + (0,) * (nd - 2))


_SPECS = {"row": _row_spec, "lead": _lead_row_spec, "full": _full_spec}


def _tc_call(body, in_arrays, out_shapes, in_kinds, out_kinds):
    in_specs = [_SPECS[k](a.shape) for a, k in zip(in_arrays, in_kinds)]
    out_specs = [_SPECS[k](s.shape) for s, k in zip(out_shapes, out_kinds)]
    single = len(out_shapes) == 1
    return pl.pallas_call(
        body,
        grid=(NBLK,),
        in_specs=in_specs,
        out_specs=out_specs[0] if single else out_specs,
        out_shape=out_shapes[0] if single else out_shapes,
    )(*in_arrays)


def kernel(x, edge_index, W1, a_src1, a_dst1, b1, g1, be1, W2, a_src2,
           a_dst2, b2, g2, be2, W3, b3, g3, be3, Wl, bl):
    f32 = _f32
    xp = jnp.zeros((NP, D_IN), f32).at[:N].set(x)
    loop = jnp.arange(N, dtype=edge_index.dtype)
    padi = jnp.full((EP - ET,), N, dtype=edge_index.dtype)
    src = jnp.concatenate([edge_index[0], loop, padi])
    dst = jnp.concatenate([edge_index[1], loop, padi])

    eye = jnp.eye(HEADS, dtype=f32)

    def head_proj(a):
        A = (eye[:, None, :] * a[:, :, None]).reshape(HD, HEADS)
        return A.reshape(NHP, 128, HEADS)

    def ab_tables(a_s, a_d):
        As = head_proj(a_s)
        Ad = head_proj(a_d)
        return (jnp.concatenate([As, Ad], axis=2),
                jnp.concatenate([Ad, As], axis=2))

    As1, Ad1 = ab_tables(a_src1, a_dst1)
    As2, Ad2 = ab_tables(a_src2, a_dst2)
    W1r = W1.reshape(D_IN, NHP, 128).transpose(1, 0, 2)
    W2r = W2.reshape(NHP, 128, NHP, 128).transpose(0, 2, 1, 3)
    W3r = W3.reshape(NHP, 128, HID)
    b1r, g1r, be1r = (v.reshape(NHP, 128) for v in (b1, g1, be1))
    b2r, g2r, be2r = (v.reshape(NHP, 128) for v in (b2, g2, be2))

    h1, ts1, td1 = _tc_call(
        _p0_body, [xp, W1r, As1, Ad1],
        [jax.ShapeDtypeStruct((NHP, NP, 128), f32),
         jax.ShapeDtypeStruct((NP, 16), f32),
         jax.ShapeDtypeStruct((NP, 16), f32)],
        ["row", "full", "full", "full"], ["lead", "row", "row"])
    src2 = src.reshape(EP // KB, KB)
    dst2 = dst.reshape(EP // KB, KB)
    w1e, den1, deg = _kw(ts1, td1, src2, dst2)
    num1 = _kagg(h1, w1e, src, dst)

    h2, ts2, td2 = _tc_call(
        _p3_body, [num1, den1, b1r, g1r, be1r, W2r, As2, Ad2],
        [jax.ShapeDtypeStruct((NHP, NP, 128), f32),
         jax.ShapeDtypeStruct((NP, 16), f32),
         jax.ShapeDtypeStruct((NP, 16), f32)],
        ["lead", "lead", "full", "full", "full", "full", "full", "full"],
        ["lead", "row", "row"])
    w2e, den2 = _kw_nodeg(ts2, td2, src2, dst2)
    num2 = _kagg(h2, w2e, src, dst)

    h3 = _tc_call(
        _p6_body, [num2, den2, b2r, g2r, be2r, W3r, deg],
        [jax.ShapeDtypeStruct((NP, HID), f32)],
        ["lead", "lead", "full", "full", "full", "full", "lead"], ["row"])
    gcn = _kgcn(h3, src2, dst2)
    out = _tc_call(
        _p8_body, [gcn, deg, b3, g3, be3, Wl, bl],
        [jax.ShapeDtypeStruct((NP, 2), f32)],
        ["lead", "lead", "full", "full", "full", "full", "full"], ["row"])
    return out[:N]


# speedup vs baseline: 1.0667x; 1.0667x over previous
import functools

import jax
import jax.numpy as jnp
from jax import lax
from jax.experimental import pallas as pl
from jax.experimental.pallas import tpu as pltpu
from jax.experimental.pallas import tpu_sc as plsc

N = 10000
E = 320000
ET = E + N
D_IN = 128
HID = 64
HEADS = 8
HD = HEADS * HID
NHP = 4
NP = 10240
EP = 331776
NC = 2
NS = 16
KB = 128
RPT = NP // NS

BN = 256
NBLK = NP // BN

def _mesh():
    return plsc.VectorSubcoreMesh(
        core_axis_name="c", subcore_axis_name="s",
        num_cores=NC, num_subcores=NS)


_SC_PARAMS = pltpu.CompilerParams(use_tc_tiling_on_sc=False,
                                  needs_layout_passes=False)


_f32 = jnp.float32


def _zero_vmem(ref, nrow, ncol):
    zv = jnp.zeros((16,), _f32)

    def body(r, _):
        for j in range(ncol // 16):
            ref[r, pl.ds(j * 16, 16)] = zv
        return 0

    lax.fori_loop(0, nrow, body, 0)


_KW_SB = 3
_KW_NSB = EP // (NC * NS) // (_KW_SB * KB)
_KW_NPAIR = _KW_NSB // 2


def _kw_core(want_deg, ts_hbm, td_hbm, src2_hbm, dst2_hbm, w_hbm, den_hbm,
             deg_hbm, is_a, id_a, is_b, id_b, rs_a, rd_a, rs_b, rd_b,
             wv_a, wv_b, ones, den_sh, deg_sh, sem_a, sem_b):
    cid = lax.axis_index("c")
    sid = lax.axis_index("s")
    row0 = sid * RPT
    SB = _KW_SB
    tile_blk0 = (cid * NS + sid) * _KW_NSB * SB

    zb = wv_a.at[0]
    _zero_vmem(zb, KB, 16)
    for i in range(RPT // KB):
        pltpu.sync_copy(zb, den_sh.at[pl.ds(row0 + i * KB, KB)])
        if want_deg:
            pltpu.sync_copy(zb, deg_sh.at[pl.ds(row0 + i * KB, KB)])
    if want_deg:
        ov = jnp.full((16,), 1.0, _f32)

        def fill1(r, _):
            ones[r, pl.ds(0, 16)] = ov
            return 0

        lax.fori_loop(0, KB, fill1, 0)
    plsc.subcore_barrier()

    def fire(s, is2, id2, rs3, rd3, sem):
        brow = tile_blk0 + s * SB
        pltpu.sync_copy(src2_hbm.at[pl.ds(brow, SB)], is2)
        pltpu.sync_copy(dst2_hbm.at[pl.ds(brow, SB)], id2)
        for j in range(SB):
            pltpu.async_copy(ts_hbm.at[is2.at[j]], rs3.at[j], sem)
            pltpu.async_copy(td_hbm.at[id2.at[j]], rd3.at[j], sem)

    def drain(is2, id2, rs3, rd3, sem):
        for j in range(SB):
            pltpu.make_async_copy(ts_hbm.at[is2.at[j]], rs3.at[j], sem).wait()
            pltpu.make_async_copy(td_hbm.at[id2.at[j]], rd3.at[j], sem).wait()

    def proc(s, id2, rs3, rd3, wv3):
        brow = tile_blk0 + s * SB
        for j in range(SB):
            rsj = rs3.at[j]
            rdj = rd3.at[j]
            wvj = wv3.at[j]

            @plsc.parallel_loop(0, KB, unroll=8)
            def _(k):
                e = rsj[k, pl.ds(0, 16)] + rdj[k, pl.ds(0, 16)]
                e = jnp.where(e > 0, e, 0.2 * e)
                wvj[k, pl.ds(0, 16)] = jnp.exp(e)

            pltpu.sync_copy(wvj, w_hbm.at[pl.ds((brow + j) * KB, KB)])
            pltpu.sync_copy(wvj, den_sh.at[id2.at[j]], add=True)
            if want_deg:
                pltpu.sync_copy(ones, deg_sh.at[id2.at[j]], add=True)

    fire(0, is_a, id_a, rs_a, rd_a, sem_a)

    def pair(p, _):
        s0 = 2 * p
        fire(s0 + 1, is_b, id_b, rs_b, rd_b, sem_b)
        drain(is_a, id_a, rs_a, rd_a, sem_a)
        proc(s0, id_a, rs_a, rd_a, wv_a)
        fire(s0 + 2, is_a, id_a, rs_a, rd_a, sem_a)
        drain(is_b, id_b, rs_b, rd_b, sem_b)
        proc(s0 + 1, id_b, rs_b, rd_b, wv_b)
        return 0

    lax.fori_loop(0, _KW_NPAIR, pair, 0)
    drain(is_a, id_a, rs_a, rd_a, sem_a)
    proc(_KW_NSB - 1, id_a, rs_a, rd_a, wv_a)

    plsc.subcore_barrier()
    pltpu.sync_copy(den_sh.at[pl.ds(row0, RPT)],
                    den_hbm.at[cid].at[pl.ds(row0, RPT)])
    if want_deg:
        pltpu.sync_copy(deg_sh.at[pl.ds(row0, RPT)],
                        deg_hbm.at[cid].at[pl.ds(row0, RPT)])


def _kw_body_deg(*args):
    _kw_core(True, *args)


def _kw_body_nodeg(ts_hbm, td_hbm, src2_hbm, dst2_hbm, w_hbm, den_hbm,
                   *rest):
    _kw_core(False, ts_hbm, td_hbm, src2_hbm, dst2_hbm, w_hbm, den_hbm,
             None, *rest)


@functools.lru_cache(maxsize=None)
def _build_kw(want_deg):
    out_type = [
        jax.ShapeDtypeStruct((EP, 16), _f32),
        jax.ShapeDtypeStruct((NC, NP, 16), _f32),
    ]
    if want_deg:
        out_type.append(jax.ShapeDtypeStruct((NC, NP, 16), _f32))
    return pl.kernel(
        _kw_body_deg if want_deg else _kw_body_nodeg,
        out_type=out_type,
        mesh=_mesh(),
        compiler_params=_SC_PARAMS,
        scratch_types=[
            pltpu.VMEM((_KW_SB, KB), jnp.int32),
            pltpu.VMEM((_KW_SB, KB), jnp.int32),
            pltpu.VMEM((_KW_SB, KB), jnp.int32),
            pltpu.VMEM((_KW_SB, KB), jnp.int32),
            pltpu.VMEM((_KW_SB, KB, 16), _f32),
            pltpu.VMEM((_KW_SB, KB, 16), _f32),
            pltpu.VMEM((_KW_SB, KB, 16), _f32),
            pltpu.VMEM((_KW_SB, KB, 16), _f32),
            pltpu.VMEM((_KW_SB, KB, 16), _f32),
            pltpu.VMEM((_KW_SB, KB, 16), _f32),
            pltpu.VMEM((KB, 16), _f32),
            pltpu.VMEM_SHARED((NP, 16), _f32),
            pltpu.VMEM_SHARED((NP, 16), _f32),
            pltpu.SemaphoreType.DMA,
            pltpu.SemaphoreType.DMA,
        ],
    )


def _kw(ts, td, src2, dst2):
    return _build_kw(True)(ts, td, src2, dst2)


def _kw_nodeg(ts, td, src2, dst2):
    return _build_kw(False)(ts, td, src2, dst2)


_AGG_NBLK = EP // NS // KB
_AGG_NPAIR = _AGG_NBLK // 2


def _kagg_body(h_hbm, w_hbm, src_hbm, dst_hbm, num_hbm,
               is_a, id_a, is_b, id_b, rows_a, rows_b, wv_a, wv_b, acc,
               sem_a, sem_b):
    cid = lax.axis_index("c")
    sid = lax.axis_index("s")
    nblk = _AGG_NBLK
    row0 = sid * RPT
    blk0 = sid * nblk

    for hpi in range(2):
        _zero_vmem(rows_a, KB, 128)
        for i in range(RPT // KB):
            pltpu.sync_copy(rows_a, acc.at[pl.ds(row0 + i * KB, KB)])
        plsc.subcore_barrier()

        h_hp = h_hbm.at[2 * cid + hpi]
        num_hp = num_hbm.at[2 * cid + hpi]
        col0 = jnp.full((16,), 4 * cid + 2 * hpi, jnp.int32)
        col1 = col0 + 1

        def fire(b, is_r, id_r, rows_r, wv_r, sem):
            base = (blk0 + b) * KB
            pltpu.sync_copy(src_hbm.at[pl.ds(base, KB)], is_r)
            pltpu.sync_copy(dst_hbm.at[pl.ds(base, KB)], id_r)
            pltpu.async_copy(h_hp.at[is_r], rows_r, sem)
            pltpu.async_copy(w_hbm.at[pl.ds(base, KB)], wv_r, sem)

        def drain(is_r, rows_r, wv_r, sem):
            pltpu.make_async_copy(h_hp.at[is_r], rows_r, sem).wait()
            pltpu.make_async_copy(w_hbm.at[pl.ds(0, KB)], wv_r, sem).wait()

        def scale(rows_r, wv_r):
            @plsc.parallel_loop(0, KB, unroll=8)
            def _(k):
                kvec = jnp.full((16,), k, jnp.int32)
                w0 = plsc.load_gather(wv_r, [kvec, col0])
                w1 = plsc.load_gather(wv_r, [kvec, col1])
                for j in range(4):
                    rows_r[k, pl.ds(j * 16, 16)] = (
                        rows_r[k, pl.ds(j * 16, 16)] * w0)
                for j in range(4, 8):
                    rows_r[k, pl.ds(j * 16, 16)] = (
                        rows_r[k, pl.ds(j * 16, 16)] * w1)

        fire(0, is_a, id_a, rows_a, wv_a, sem_a)

        def pair(p, _):
            b0 = 2 * p
            fire(b0 + 1, is_b, id_b, rows_b, wv_b, sem_b)
            drain(is_a, rows_a, wv_a, sem_a)
            scale(rows_a, wv_a)
            pltpu.sync_copy(rows_a, acc.at[id_a], add=True)

            @pl.when(p < _AGG_NPAIR - 1)
            def _():
                fire(b0 + 2, is_a, id_a, rows_a, wv_a, sem_a)

            drain(is_b, rows_b, wv_b, sem_b)
            scale(rows_b, wv_b)
            pltpu.sync_copy(rows_b, acc.at[id_b], add=True)
            return 0

        lax.fori_loop(0, _AGG_NPAIR, pair, 0)
        plsc.subcore_barrier()
        pltpu.sync_copy(acc.at[pl.ds(row0, RPT)], num_hp.at[pl.ds(row0, RPT)])
        plsc.subcore_barrier()


@functools.lru_cache(maxsize=None)
def _build_kagg():
    return pl.kernel(
        _kagg_body,
        out_type=jax.ShapeDtypeStruct((NHP, NP, 128), _f32),
        mesh=_mesh(),
        compiler_params=_SC_PARAMS,
        scratch_types=[
            pltpu.VMEM((KB,), jnp.int32),
            pltpu.VMEM((KB,), jnp.int32),
            pltpu.VMEM((KB,), jnp.int32),
            pltpu.VMEM((KB,), jnp.int32),
            pltpu.VMEM((KB, 128), _f32),
            pltpu.VMEM((KB, 128), _f32),
            pltpu.VMEM((KB, 16), _f32),
            pltpu.VMEM((KB, 16), _f32),
            pltpu.VMEM_SHARED((NP, 128), _f32),
            pltpu.SemaphoreType.DMA,
            pltpu.SemaphoreType.DMA,
        ],
    )


def _kagg(*args):
    return _build_kagg()(*args)


def _kgcn_body(h_hbm, src2_hbm, dst2_hbm, out_hbm,
               is_a, id_a, is_b, id_b, rows_a, rows_b, acc, sem_a, sem_b):
    cid = lax.axis_index("c")
    sid = lax.axis_index("s")
    row0 = sid * RPT
    SB = _KW_SB
    tile_blk0 = (cid * NS + sid) * _KW_NSB * SB

    zb = rows_a.at[0]
    _zero_vmem(zb, KB, 64)
    for i in range(RPT // KB):
        pltpu.sync_copy(zb, acc.at[pl.ds(row0 + i * KB, KB)])
    plsc.subcore_barrier()

    def fire(s, is2, id2, rows3, sem):
        brow = tile_blk0 + s * SB
        pltpu.sync_copy(src2_hbm.at[pl.ds(brow, SB)], is2)
        pltpu.sync_copy(dst2_hbm.at[pl.ds(brow, SB)], id2)
        for j in range(SB):
            pltpu.async_copy(h_hbm.at[is2.at[j]], rows3.at[j], sem)

    def drain(is2, rows3, sem):
        for j in range(SB):
            pltpu.make_async_copy(h_hbm.at[is2.at[j]], rows3.at[j],
                                  sem).wait()

    def proc(id2, rows3):
        for j in range(SB):
            pltpu.sync_copy(rows3.at[j], acc.at[id2.at[j]], add=True)

    fire(0, is_a, id_a, rows_a, sem_a)

    def pair(p, _):
        fire(2 * p + 1, is_b, id_b, rows_b, sem_b)
        drain(is_a, rows_a, sem_a)
        proc(id_a, rows_a)
        fire(2 * p + 2, is_a, id_a, rows_a, sem_a)
        drain(is_b, rows_b, sem_b)
        proc(id_b, rows_b)
        return 0

    lax.fori_loop(0, _KW_NPAIR, pair, 0)
    drain(is_a, rows_a, sem_a)
    proc(id_a, rows_a)

    plsc.subcore_barrier()
    pltpu.sync_copy(acc.at[pl.ds(row0, RPT)],
                    out_hbm.at[cid].at[pl.ds(row0, RPT)])


@functools.lru_cache(maxsize=None)
def _build_kgcn():
    return pl.kernel(
        _kgcn_body,
        out_type=jax.ShapeDtypeStruct((NC, NP, 64), _f32),
        mesh=_mesh(),
        compiler_params=_SC_PARAMS,
        scratch_types=[
            pltpu.VMEM((_KW_SB, KB), jnp.int32),
            pltpu.VMEM((_KW_SB, KB), jnp.int32),
            pltpu.VMEM((_KW_SB, KB), jnp.int32),
            pltpu.VMEM((_KW_SB, KB), jnp.int32),
            pltpu.VMEM((_KW_SB, KB, 64), _f32),
            pltpu.VMEM((_KW_SB, KB, 64), _f32),
            pltpu.VMEM_SHARED((NP, 64), _f32),
            pltpu.SemaphoreType.DMA,
            pltpu.SemaphoreType.DMA,
        ],
    )


def _kgcn(*args):
    return _build_kgcn()(*args)


_DOT = functools.partial(jnp.dot, preferred_element_type=_f32)


def _p0_body(x_ref, w1_ref, as_ref, ad_ref, h_ref, ts_ref, td_ref):
    xb = x_ref[...]
    hs = []
    for hp in range(NHP):
        h = _DOT(xb, w1_ref[:, hp, :])
        h_ref[hp] = h
        hs.append(h)
    ts_ref[...] = sum(_DOT(hs[hp], as_ref[hp]) for hp in range(NHP))
    td_ref[...] = sum(_DOT(hs[hp], ad_ref[hp]) for hp in range(NHP))


def _norm_elu(num_ref, den_ref, b_ref, g_ref, be_ref, hp):
    den = den_ref[0] + den_ref[1]
    r0 = 1.0 / (den[:, 2 * hp:2 * hp + 1] + 1e-16)
    r1 = 1.0 / (den[:, 2 * hp + 1:2 * hp + 2] + 1e-16)
    numb = num_ref[hp]
    y = jnp.concatenate([numb[:, 0:64] * r0, numb[:, 64:128] * r1], axis=1)
    y = y + b_ref[hp][None, :]
    y = y * (g_ref[hp][None, :] / jnp.sqrt(1.0 + 1e-5)) + be_ref[hp][None, :]
    return jnp.where(y > 0, y, jnp.exp(y) - 1.0)


def _p3_body(num_ref, den_ref, b_ref, g_ref, be_ref, w2_ref, as_ref, ad_ref,
             h2_ref, ts_ref, td_ref):
    ys = [_norm_elu(num_ref, den_ref, b_ref, g_ref, be_ref, hp)
          for hp in range(NHP)]
    for ohp in range(NHP):
        h2_ref[ohp] = sum(_DOT(ys[hp], w2_ref[hp, :, ohp, :])
                          for hp in range(NHP))
    ts_ref[...] = sum(_DOT(ys[hp], as_ref[hp]) for hp in range(NHP))
    td_ref[...] = sum(_DOT(ys[hp], ad_ref[hp]) for hp in range(NHP))


def _p6_body(num_ref, den_ref, b_ref, g_ref, be_ref, w3_ref, deg_ref, h3_ref):
    ys = [_norm_elu(num_ref, den_ref, b_ref, g_ref, be_ref, hp)
          for hp in range(NHP)]
    h3 = sum(_DOT(ys[hp], w3_ref[hp]) for hp in range(NHP))
    deg = (deg_ref[0] + deg_ref[1])[:, 0:1]
    dinv = jax.lax.rsqrt(jnp.maximum(deg, 1.0))
    h3_ref[...] = h3 * dinv


def _p8_body(gcn_ref, deg_ref, b3_ref, g3_ref, be3_ref, wl_ref, bl_ref,
             out_ref):
    deg = (deg_ref[0] + deg_ref[1])[:, 0:1]
    dinv = jax.lax.rsqrt(jnp.maximum(deg, 1.0))
    y = (gcn_ref[0] + gcn_ref[1]) * dinv + b3_ref[...][None, :]
    y = y * (g3_ref[...][None, :] / jnp.sqrt(1.0 + 1e-5)) + be3_ref[...][None, :]
    y = jnp.where(y > 0, y, jnp.exp(y) - 1.0)
    logits = _DOT(y, wl_ref[...]) + bl_ref[...][None, :]
    l0 = logits[:, 0:1]
    l1 = logits[:, 1:2]
    m = jnp.maximum(l0, l1)
    lse = m + jnp.log(jnp.exp(l0 - m) + jnp.exp(l1 - m))
    out_ref[...] = jnp.concatenate([l0 - lse, l1 - lse], axis=1)


def _row_spec(shape):
    nd = len(shape)
    return pl.BlockSpec((BN,) + shape[1:], lambda i: (i,) + (0,) * (nd - 1))


def _full_spec(shape):
    nd = len(shape)
    return pl.BlockSpec(shape, lambda i: (0,) * nd)


def _lead_row_spec(shape):
    nd = len(shape)
    return pl.BlockSpec((shape[0], BN) + shape[2:],
                        lambda i: (0, i) + (0,) * (nd - 2))


_SPECS = {"row": _row_spec, "lead": _lead_row_spec, "full": _full_spec}


def _tc_call(body, in_arrays, out_shapes, in_kinds, out_kinds):
    in_specs = [_SPECS[k](a.shape) for a, k in zip(in_arrays, in_kinds)]
    out_specs = [_SPECS[k](s.shape) for s, k in zip(out_shapes, out_kinds)]
    single = len(out_shapes) == 1
    return pl.pallas_call(
        body,
        grid=(NBLK,),
        in_specs=in_specs,
        out_specs=out_specs[0] if single else out_specs,
        out_shape=out_shapes[0] if single else out_shapes,
    )(*in_arrays)


def kernel(x, edge_index, W1, a_src1, a_dst1, b1, g1, be1, W2, a_src2,
           a_dst2, b2, g2, be2, W3, b3, g3, be3, Wl, bl):
    f32 = _f32
    xp = jnp.zeros((NP, D_IN), f32).at[:N].set(x)
    loop = jnp.arange(N, dtype=edge_index.dtype)
    padi = jnp.full((EP - ET,), N, dtype=edge_index.dtype)
    src = jnp.concatenate([edge_index[0], loop, padi])
    dst = jnp.concatenate([edge_index[1], loop, padi])

    eye = jnp.eye(HEADS, dtype=f32)

    def head_proj(a):
        A = (eye[:, None, :] * a[:, :, None]).reshape(HD, HEADS)
        return A.reshape(NHP, 128, HEADS)

    def ab_tables(a_s, a_d):
        As = head_proj(a_s)
        Ad = head_proj(a_d)
        return (jnp.concatenate([As, Ad], axis=2),
                jnp.concatenate([Ad, As], axis=2))

    As1, Ad1 = ab_tables(a_src1, a_dst1)
    As2, Ad2 = ab_tables(a_src2, a_dst2)
    W1r = W1.reshape(D_IN, NHP, 128)
    W2r = W2.reshape(NHP, 128, NHP, 128)
    W3r = W3.reshape(NHP, 128, HID)
    b1r, g1r, be1r = (v.reshape(NHP, 128) for v in (b1, g1, be1))
    b2r, g2r, be2r = (v.reshape(NHP, 128) for v in (b2, g2, be2))

    h1, ts1, td1 = _tc_call(
        _p0_body, [xp, W1r, As1, Ad1],
        [jax.ShapeDtypeStruct((NHP, NP, 128), f32),
         jax.ShapeDtypeStruct((NP, 16), f32),
         jax.ShapeDtypeStruct((NP, 16), f32)],
        ["row", "full", "full", "full"], ["lead", "row", "row"])
    src2 = src.reshape(EP // KB, KB)
    dst2 = dst.reshape(EP // KB, KB)
    w1e, den1, deg = _kw(ts1, td1, src2, dst2)
    num1 = _kagg(h1, w1e, src, dst)

    h2, ts2, td2 = _tc_call(
        _p3_body, [num1, den1, b1r, g1r, be1r, W2r, As2, Ad2],
        [jax.ShapeDtypeStruct((NHP, NP, 128), f32),
         jax.ShapeDtypeStruct((NP, 16), f32),
         jax.ShapeDtypeStruct((NP, 16), f32)],
        ["lead", "lead", "full", "full", "full", "full", "full", "full"],
        ["lead", "row", "row"])
    w2e, den2 = _kw_nodeg(ts2, td2, src2, dst2)
    num2 = _kagg(h2, w2e, src, dst)

    h3 = _tc_call(
        _p6_body, [num2, den2, b2r, g2r, be2r, W3r, deg],
        [jax.ShapeDtypeStruct((NP, HID), f32)],
        ["lead", "lead", "full", "full", "full", "full", "lead"], ["row"])
    gcn = _kgcn(h3, src2, dst2)
    out = _tc_call(
        _p8_body, [gcn, deg, b3, g3, be3, Wl, bl],
        [jax.ShapeDtypeStruct((NP, 2), f32)],
        ["lead", "lead", "full", "full", "full", "full", "full"], ["row"])
    return out[:N]
